# baseline (speedup 1.0000x reference)
import functools

import jax
import jax.numpy as jnp
from jax import lax
from jax.experimental import pallas as pl
from jax.experimental.pallas import tpu as pltpu
from jax.experimental.pallas import tpu_sc as plsc

LANES = 16


def _make_sc_pool(batch, seq, emb, n_workers):
    bpw = batch // n_workers
    mesh = plsc.VectorSubcoreMesh(core_axis_name="c", subcore_axis_name="s")
    n_cores = 2
    nj = emb // LANES

    @functools.partial(
        pl.kernel,
        mesh=mesh,
        out_type=jax.ShapeDtypeStruct((batch, emb), jnp.float32),
        scratch_types=[
            pltpu.VMEM((bpw, seq), jnp.int32),
            pltpu.VMEM((seq, emb), jnp.float32),
            pltpu.VMEM((seq, emb), jnp.float32),
            pltpu.VMEM((bpw, emb), jnp.float32),
            pltpu.SemaphoreType.DMA,
            pltpu.SemaphoreType.DMA,
        ],
    )
    def sc_pool(idx_hbm, table_hbm, out_hbm, idx_v, rows0, rows1, out_v, sem0, sem1):
        wid = lax.axis_index("s") * n_cores + lax.axis_index("c")
        base = wid * bpw
        pltpu.sync_copy(idx_hbm.at[pl.ds(base, bpw)], idx_v)

        rows = (rows0, rows1)
        sems = (sem0, sem1)

        def fire(bb, rbuf, sem):
            def body(i, _):
                vec = idx_v[bb, pl.ds(i * LANES, LANES)]
                for k in range(LANES):
                    s = i * LANES + k
                    pltpu.async_copy(
                        table_hbm.at[pl.ds(vec[k], 1), :],
                        rbuf.at[pl.ds(s, 1), :],
                        sem,
                    )
                return 0

            lax.fori_loop(0, seq // LANES, body, 0)
            rem = seq % LANES
            if rem:
                vec = idx_v[bb, pl.ds(seq - LANES, LANES)]
                for k in range(LANES - rem, LANES):
                    s = seq - LANES + k
                    pltpu.async_copy(
                        table_hbm.at[pl.ds(vec[k], 1), :],
                        rbuf.at[pl.ds(s, 1), :],
                        sem,
                    )

        def wait_fetch(rbuf, sem):
            pltpu.make_async_copy(
                table_hbm.at[pl.ds(0, seq), :], rbuf, sem
            ).wait()

        def reduce_rows(rbuf, b):
            def inner(i, accs):
                new = list(accs)
                for k in range(8):
                    s8 = i * 8 + k
                    for j in range(nj):
                        new[j] = new[j] + rbuf[s8, pl.ds(j * LANES, LANES)]
                return tuple(new)

            accs = tuple(jnp.zeros((LANES,), jnp.float32) for _ in range(nj))
            accs = lax.fori_loop(0, seq // 8, inner, accs)
            for j in range(nj):
                out_v[b, pl.ds(j * LANES, LANES)] = accs[j]

        fire(0, rows0, sem0)

        def outer(b2, _):
            b = b2 * 2
            for p in range(2):
                bb = b + p

                @pl.when(bb + 1 < bpw)
                def _():
                    fire(bb + 1, rows[1 - p], sems[1 - p])

                wait_fetch(rows[p], sems[p])
                reduce_rows(rows[p], bb)
            return 0

        lax.fori_loop(0, bpw // 2, outer, 0)
        pltpu.sync_copy(out_v, out_hbm.at[pl.ds(base, bpw)])

    return sc_pool


def _dense_body(inv_seq, x_ref, w1_ref, b1_ref, w2_ref, b2_ref, o_ref):
    x = x_ref[...] * inv_seq
    h = jnp.dot(x, w1_ref[...], preferred_element_type=jnp.float32) + b1_ref[...]
    h = jnp.maximum(h, 0.0)
    logits = jnp.dot(h, w2_ref[...], preferred_element_type=jnp.float32) + b2_ref[...]
    m = jnp.max(logits, axis=-1, keepdims=True)
    e = jnp.exp(logits - m)
    o_ref[...] = e / jnp.sum(e, axis=-1, keepdims=True)


def kernel(inputs, emb_table, W1, b1, W2, b2):
    batch, seq = inputs.shape
    vocab, emb = emb_table.shape
    n_classes = W2.shape[1]

    idx = inputs.astype(jnp.int32)
    sc_pool = _make_sc_pool(batch, seq, emb, 32)
    pooled = sc_pool(idx, emb_table)

    dense = pl.pallas_call(
        functools.partial(_dense_body, 1.0 / seq),
        out_shape=jax.ShapeDtypeStruct((batch, n_classes), jnp.float32),
    )
    return dense(pooled, W1, b1.reshape(1, -1), W2, b2.reshape(1, -1))
